# baseline (speedup 1.0000x reference)
import jax
import jax.numpy as jnp
from jax import lax
from jax.experimental import pallas as pl
from jax.experimental.pallas import tpu as pltpu
from jax.experimental.pallas import tpu_sc as plsc

B, C, H, W = 4, 384, 224, 224
ROW = H * W
R = B * C
QUARTS = 4
CH = ROW // QUARTS

_info = plsc.get_sparse_core_info()
_NC = _info.num_cores
_NS = _info.num_subcores
NW = _NC * _NS
RPW = R // NW
NCHUNK = RPW * QUARTS
NBUF = 8


def _sc_body(in_hbm, out_hbm, *rest):
    bufs = rest[:NBUF]
    gsem = rest[NBUF:2 * NBUF]
    ssem = rest[2 * NBUF:3 * NBUF]

    wid = lax.axis_index("s") * _NC + lax.axis_index("c")
    base = wid * RPW
    b = base // C
    src0 = 2 * b * C + (C - 1) - base

    def src_chunk(t):
        return (src0 - (t // QUARTS)) * QUARTS + (t % QUARTS)

    def dst_chunk(t):
        return (base + (t // QUARTS)) * QUARTS + (t % QUARTS)

    for k in range(NBUF):
        pltpu.async_copy(in_hbm.at[src_chunk(k)], bufs[k], gsem[k])

    @pl.loop(0, NCHUNK, step=NBUF)
    def _(t0):
        for k in range(NBUF):
            pltpu.make_async_copy(in_hbm.at[src_chunk(t0 + k)], bufs[k],
                                  gsem[k]).wait()
            pltpu.async_copy(bufs[k], out_hbm.at[dst_chunk(t0 + k)], ssem[k])
        for k in range(NBUF):
            pltpu.make_async_copy(bufs[k], out_hbm.at[dst_chunk(t0 + k)],
                                  ssem[k]).wait()

            @pl.when(t0 + k + NBUF < NCHUNK)
            def _():
                pltpu.async_copy(in_hbm.at[src_chunk(t0 + k + NBUF)], bufs[k],
                                 gsem[k])


_sc_kernel = pl.kernel(
    _sc_body,
    out_type=jax.ShapeDtypeStruct((R * QUARTS, CH), jnp.float32),
    mesh=plsc.VectorSubcoreMesh(core_axis_name="c", subcore_axis_name="s"),
    scratch_types=(
        [pltpu.VMEM((CH,), jnp.float32) for _ in range(NBUF)]
        + [pltpu.SemaphoreType.DMA for _ in range(2 * NBUF)]
    ),
)


@jax.jit
def kernel(input):
    flat = input.reshape(R * QUARTS, CH)
    out = _sc_kernel(flat)
    return out.reshape(B, C, H, W)
